# Initial kernel scaffold; baseline (speedup 1.0000x reference)
import jax
import jax.numpy as jnp
from jax.experimental import pallas as pl


def kernel(w1, w2, embedding, fc_w, fc_b):
    raise NotImplementedError("write your pallas kernel here")


# speedup vs baseline: 4.8590x; 4.8590x over previous
import functools

import jax
import jax.numpy as jnp
from jax import lax
from jax.experimental import pallas as pl
from jax.experimental.pallas import tpu as pltpu
from jax.experimental.pallas import tpu_sc as plsc

_VOCAB = 10
_DIM = 4
_LANES = 16
_NC = 2
_NS = 16
_NW = _NC * _NS


def _body(emb_hbm, fc_hbm, w1_hbm, w2_hbm, out_hbm,
          emb_v, fc_v, s1_v, s2_v, w1_v, w2_v, o_v, chunk):
    wid = lax.axis_index("s") * _NC + lax.axis_index("c")
    base = wid * chunk

    pltpu.sync_copy(emb_hbm, emb_v)
    pltpu.sync_copy(fc_hbm, fc_v)
    pltpu.sync_copy(w1_hbm.at[pl.ds(base, chunk)], w1_v)
    pltpu.sync_copy(w2_hbm.at[pl.ds(base, chunk)], w2_v)

    lanes4 = lax.iota(jnp.int32, _LANES) * _DIM
    fcv = fc_v[...]
    s1 = jnp.zeros((_LANES,), jnp.float32)
    s2 = jnp.zeros((_LANES,), jnp.float32)
    for d in range(_DIM):
        col = plsc.load_gather(emb_v, [lanes4 + d])
        s1 = s1 + col * fcv[d]
        s2 = s2 + col * fcv[_DIM + d]
    bias = fcv[2 * _DIM]
    s1_v[...] = s1
    s2_v[...] = s2

    for j in range(chunk // _LANES):
        sl = pl.ds(j * _LANES, _LANES)
        a = (plsc.load_gather(s1_v, [w1_v[sl]])
             + plsc.load_gather(s2_v, [w2_v[sl]]) + bias)
        o_v[sl] = 1.0 / (1.0 + jnp.exp(-a))

    pltpu.sync_copy(o_v, out_hbm.at[pl.ds(base, chunk)])


def kernel(w1, w2, embedding, fc_w, fc_b):
    batch = w1.shape[0]
    chunk = batch // _NW

    emb_flat = jnp.pad(embedding, ((0, _LANES - _VOCAB), (0, 0))).reshape(-1)
    fc_pack = jnp.zeros((_LANES,), jnp.float32)
    fc_pack = fc_pack.at[: 2 * _DIM].set(fc_w[0])
    fc_pack = fc_pack.at[2 * _DIM].set(fc_b[0])

    mesh = plsc.VectorSubcoreMesh(
        core_axis_name="c", subcore_axis_name="s",
        num_cores=_NC, num_subcores=_NS)
    run = pl.kernel(
        functools.partial(_body, chunk=chunk),
        out_type=jax.ShapeDtypeStruct((batch,), jnp.float32),
        mesh=mesh,
        compiler_params=pltpu.CompilerParams(needs_layout_passes=False),
        scratch_types=[
            pltpu.VMEM((_LANES * _DIM,), jnp.float32),
            pltpu.VMEM((_LANES,), jnp.float32),
            pltpu.VMEM((_LANES,), jnp.float32),
            pltpu.VMEM((_LANES,), jnp.float32),
            pltpu.VMEM((chunk,), jnp.int32),
            pltpu.VMEM((chunk,), jnp.int32),
            pltpu.VMEM((chunk,), jnp.float32),
        ],
    )
    return run(emb_flat, fc_pack, w1, w2)
